# baseline (speedup 1.0000x reference)
import functools

import jax
import jax.numpy as jnp
from jax import lax
from jax.experimental import pallas as pl
from jax.experimental.pallas import tpu as pltpu
from jax.experimental.pallas import tpu_sc as plsc

D = 128
NC = 2
NS = 16
NW = NC * NS
CHUNK = 128
UNIT = NW * CHUNK
BLOCK = 2048
PIECES = (2, 1, 1)


def _gather_body(u0, k, table_hbm, idx_hbm, out_hbm, idx_v, rows_v,
                 g_sem0, g_sem1, w_sem0, w_sem1):
    wid = lax.axis_index("s") * NC + lax.axis_index("c")
    for j in range(k):
        pltpu.sync_copy(idx_hbm.at[u0 + j].at[wid], idx_v.at[j])
    g_sems = [g_sem0, g_sem1]
    w_sems = [w_sem0, w_sem1]

    def fire(j):
        return pltpu.async_copy(
            table_hbm.at[idx_v.at[j]], rows_v.at[j % 2], g_sems[j % 2]
        )

    inflight = {0: fire(0)}
    if k > 1:
        inflight[1] = fire(1)
    writes = {}
    for j in range(k):
        inflight.pop(j).wait()
        writes[j] = pltpu.async_copy(
            rows_v.at[j % 2], out_hbm.at[j].at[wid], w_sems[j % 2]
        )
        if j + 2 < k:
            writes.pop(j).wait()
            inflight[j + 2] = fire(j + 2)
    for w in writes.values():
        w.wait()


def _sc_gather(table, idx3, u0, k):
    mesh = plsc.VectorSubcoreMesh(
        core_axis_name="c", subcore_axis_name="s", num_cores=NC, num_subcores=NS
    )
    return pl.kernel(
        functools.partial(_gather_body, u0, k),
        out_type=jax.ShapeDtypeStruct((k, NW, CHUNK, D), jnp.float32),
        mesh=mesh,
        scratch_types=[
            pltpu.VMEM((k, CHUNK), jnp.int32),
            pltpu.VMEM((2, CHUNK, D), jnp.float32),
            pltpu.SemaphoreType.DMA,
            pltpu.SemaphoreType.DMA,
            pltpu.SemaphoreType.DMA,
            pltpu.SemaphoreType.DMA,
        ],
    )(table, idx3)


def _silu_mm_body(h_ref, w_ref, b_ref, o_ref):
    h = h_ref[...]
    h = h * jax.nn.sigmoid(h)
    o_ref[...] = (
        lax.dot_general(h, w_ref[...], (((1,), (1,)), ((), ())),
                        preferred_element_type=jnp.float32)
        + b_ref[...]
    )


def _silu_mm_body_alias(h_ref, w_ref, b_ref, y_ref, o_ref):
    del y_ref
    _silu_mm_body(h_ref, w_ref, b_ref, o_ref)


def _tc_piece(gathered, W, b2, total_batch, block_off, y=None):
    rows = gathered.shape[0]
    nb = rows // BLOCK
    in_specs = [
        pl.BlockSpec((BLOCK, D), lambda i: (i, 0)),
        pl.BlockSpec((D, D), lambda i: (0, 0)),
        pl.BlockSpec((1, D), lambda i: (0, 0)),
    ]
    args = [gathered, W, b2]
    body = _silu_mm_body
    kwargs = {}
    if y is not None:
        in_specs.append(pl.BlockSpec(memory_space=pl.ANY))
        args.append(y)
        body = _silu_mm_body_alias
        kwargs["input_output_aliases"] = {3: 0}
    return pl.pallas_call(
        body,
        out_shape=jax.ShapeDtypeStruct((total_batch, D), jnp.float32),
        grid=(nb,),
        in_specs=in_specs,
        out_specs=pl.BlockSpec((BLOCK, D), lambda i: (i + block_off, 0)),
        **kwargs,
    )(*args)


def kernel(x, emb_table, W, b):
    batch = x.shape[0]
    units = batch // UNIT
    assert sum(PIECES) == units
    idx3 = x.reshape(units, NW, CHUNK)
    b2 = b.reshape(1, D)
    gs = []
    u0 = 0
    for k in PIECES:
        gs.append(_sc_gather(emb_table, idx3, u0, k).reshape(k * UNIT, D))
        u0 += k
    y = None
    u0 = 0
    for g, k in zip(gs, PIECES):
        y = _tc_piece(g, W, b2, batch, u0 * UNIT // BLOCK, y)
        u0 += k
    return y


# speedup vs baseline: 1.0885x; 1.0885x over previous
import functools

import jax
import jax.numpy as jnp
from jax import lax
from jax.experimental import pallas as pl
from jax.experimental.pallas import tpu as pltpu
from jax.experimental.pallas import tpu_sc as plsc

D = 128
NC = 2
NS = 16
NW = NC * NS
CHUNK = 128
UNIT = NW * CHUNK
BLOCK = 4096
PIECES = (2, 2)


def _gather_body(u0, k, table_hbm, idx_hbm, out_hbm, idx_v, rows_v,
                 g_sem0, g_sem1, w_sem0, w_sem1):
    wid = lax.axis_index("s") * NC + lax.axis_index("c")
    for j in range(k):
        pltpu.sync_copy(idx_hbm.at[u0 + j].at[wid], idx_v.at[j])
    g_sems = [g_sem0, g_sem1]
    w_sems = [w_sem0, w_sem1]

    def fire(j):
        return pltpu.async_copy(
            table_hbm.at[idx_v.at[j]], rows_v.at[j % 2], g_sems[j % 2]
        )

    inflight = {0: fire(0)}
    if k > 1:
        inflight[1] = fire(1)
    writes = {}
    for j in range(k):
        inflight.pop(j).wait()
        writes[j] = pltpu.async_copy(
            rows_v.at[j % 2], out_hbm.at[j].at[wid], w_sems[j % 2]
        )
        if j + 2 < k:
            writes.pop(j).wait()
            inflight[j + 2] = fire(j + 2)
    for w in writes.values():
        w.wait()


def _sc_gather(table, idx3, u0, k):
    mesh = plsc.VectorSubcoreMesh(
        core_axis_name="c", subcore_axis_name="s", num_cores=NC, num_subcores=NS
    )
    return pl.kernel(
        functools.partial(_gather_body, u0, k),
        out_type=jax.ShapeDtypeStruct((k, NW, CHUNK, D), jnp.float32),
        mesh=mesh,
        scratch_types=[
            pltpu.VMEM((k, CHUNK), jnp.int32),
            pltpu.VMEM((2, CHUNK, D), jnp.float32),
            pltpu.SemaphoreType.DMA,
            pltpu.SemaphoreType.DMA,
            pltpu.SemaphoreType.DMA,
            pltpu.SemaphoreType.DMA,
        ],
    )(table, idx3)


def _silu_mm_body(h_ref, w_ref, b_ref, o_ref):
    h = h_ref[...]
    h = h * jax.nn.sigmoid(h)
    o_ref[...] = (
        lax.dot_general(h, w_ref[...], (((1,), (1,)), ((), ())),
                        preferred_element_type=jnp.float32)
        + b_ref[...]
    )


def _silu_mm_body_alias(h_ref, w_ref, b_ref, y_ref, o_ref):
    del y_ref
    _silu_mm_body(h_ref, w_ref, b_ref, o_ref)


def _tc_piece(gathered, W, b2, total_batch, block_off, y=None):
    rows = gathered.shape[0]
    nb = rows // BLOCK
    in_specs = [
        pl.BlockSpec((BLOCK, D), lambda i: (i, 0)),
        pl.BlockSpec((D, D), lambda i: (0, 0)),
        pl.BlockSpec((1, D), lambda i: (0, 0)),
    ]
    args = [gathered, W, b2]
    body = _silu_mm_body
    kwargs = {}
    if y is not None:
        in_specs.append(pl.BlockSpec(memory_space=pl.ANY))
        args.append(y)
        body = _silu_mm_body_alias
        kwargs["input_output_aliases"] = {3: 0}
    return pl.pallas_call(
        body,
        out_shape=jax.ShapeDtypeStruct((total_batch, D), jnp.float32),
        grid=(nb,),
        in_specs=in_specs,
        out_specs=pl.BlockSpec((BLOCK, D), lambda i: (i + block_off, 0)),
        **kwargs,
    )(*args)


def kernel(x, emb_table, W, b):
    batch = x.shape[0]
    units = batch // UNIT
    assert sum(PIECES) == units
    idx3 = x.reshape(units, NW, CHUNK)
    b2 = b.reshape(1, D)
    gs = []
    u0 = 0
    for k in PIECES:
        gs.append(_sc_gather(emb_table, idx3, u0, k).reshape(k * UNIT, D))
        u0 += k
    y = None
    u0 = 0
    for g, k in zip(gs, PIECES):
        y = _tc_piece(g, W, b2, batch, u0 * UNIT // BLOCK, y)
        u0 += k
    return y


# speedup vs baseline: 1.1194x; 1.0284x over previous
import functools

import jax
import jax.numpy as jnp
from jax import lax
from jax.experimental import pallas as pl
from jax.experimental.pallas import tpu as pltpu
from jax.experimental.pallas import tpu_sc as plsc

D = 128
NC = 2
NS = 16
NW = NC * NS
CHUNK = 128
UNIT = NW * CHUNK
BLOCK = 4096
PIECES = (2, 2)


def _gather_body(u0, k, table_hbm, idx_hbm, out_hbm, idx_v, rows_v,
                 g_sem0, g_sem1, w_sem0, w_sem1):
    wid = lax.axis_index("s") * NC + lax.axis_index("c")
    g_sems = [g_sem0, g_sem1]
    w_sems = [w_sem0, w_sem1]
    idx_copies = [
        pltpu.async_copy(idx_hbm.at[u0 + j].at[wid], idx_v.at[j],
                         w_sems[j % 2])
        for j in range(k)
    ]

    def fire(j):
        idx_copies[j].wait()
        return pltpu.async_copy(
            table_hbm.at[idx_v.at[j]], rows_v.at[j % 2], g_sems[j % 2]
        )

    inflight = {0: fire(0)}
    if k > 1:
        inflight[1] = fire(1)
    writes = {}
    for j in range(k):
        inflight.pop(j).wait()
        writes[j] = pltpu.async_copy(
            rows_v.at[j % 2], out_hbm.at[j].at[wid], w_sems[j % 2]
        )
        if j + 2 < k:
            writes.pop(j).wait()
            inflight[j + 2] = fire(j + 2)
    for w in writes.values():
        w.wait()


def _sc_gather(table, idx3, u0, k):
    mesh = plsc.VectorSubcoreMesh(
        core_axis_name="c", subcore_axis_name="s", num_cores=NC, num_subcores=NS
    )
    return pl.kernel(
        functools.partial(_gather_body, u0, k),
        out_type=jax.ShapeDtypeStruct((k, NW, CHUNK, D), jnp.float32),
        mesh=mesh,
        scratch_types=[
            pltpu.VMEM((k, CHUNK), jnp.int32),
            pltpu.VMEM((2, CHUNK, D), jnp.float32),
            pltpu.SemaphoreType.DMA,
            pltpu.SemaphoreType.DMA,
            pltpu.SemaphoreType.DMA,
            pltpu.SemaphoreType.DMA,
        ],
    )(table, idx3)


def _silu_mm_body(h_ref, w_ref, b_ref, o_ref):
    h = h_ref[...]
    h = h * jax.nn.sigmoid(h)
    o_ref[...] = (
        lax.dot_general(h, w_ref[...], (((1,), (1,)), ((), ())),
                        preferred_element_type=jnp.float32)
        + b_ref[...]
    )


def _silu_mm_body_alias(h_ref, w_ref, b_ref, y_ref, o_ref):
    del y_ref
    _silu_mm_body(h_ref, w_ref, b_ref, o_ref)


def _tc_piece(gathered, W, b2, total_batch, block_off, y=None):
    rows = gathered.shape[0]
    nb = rows // BLOCK
    in_specs = [
        pl.BlockSpec((BLOCK, D), lambda i: (i, 0)),
        pl.BlockSpec((D, D), lambda i: (0, 0)),
        pl.BlockSpec((1, D), lambda i: (0, 0)),
    ]
    args = [gathered, W, b2]
    body = _silu_mm_body
    kwargs = {}
    if y is not None:
        in_specs.append(pl.BlockSpec(memory_space=pl.ANY))
        args.append(y)
        body = _silu_mm_body_alias
        kwargs["input_output_aliases"] = {3: 0}
    return pl.pallas_call(
        body,
        out_shape=jax.ShapeDtypeStruct((total_batch, D), jnp.float32),
        grid=(nb,),
        in_specs=in_specs,
        out_specs=pl.BlockSpec((BLOCK, D), lambda i: (i + block_off, 0)),
        **kwargs,
    )(*args)


def kernel(x, emb_table, W, b):
    batch = x.shape[0]
    units = batch // UNIT
    assert sum(PIECES) == units
    idx3 = x.reshape(units, NW, CHUNK)
    b2 = b.reshape(1, D)
    gs = []
    u0 = 0
    for k in PIECES:
        gs.append(_sc_gather(emb_table, idx3, u0, k).reshape(k * UNIT, D))
        u0 += k
    y = None
    u0 = 0
    for g, k in zip(gs, PIECES):
        y = _tc_piece(g, W, b2, batch, u0 * UNIT // BLOCK, y)
        u0 += k
    return y
